# Initial kernel scaffold; baseline (speedup 1.0000x reference)
import jax
import jax.numpy as jnp
from jax.experimental import pallas as pl


def kernel(slices, targets):
    raise NotImplementedError("write your pallas kernel here")


# speedup vs baseline: 10.5678x; 10.5678x over previous
import jax
import jax.numpy as jnp
from jax.experimental import pallas as pl
from jax.experimental.pallas import tpu as pltpu

_B, _C, _H, _W = 8, 19, 224, 224
_BF = 0.1
_IGN = 255
_LS = 0.2
_MAXN = 1.0 / 100.0
_CLIP = 20.0
_XR = [1, -1, 0, 0, -1, 1, -1, 1, 0]
_YR = [0, 0, -1, 1, 1, 1, -1, -1, 0]
_INF = 1e9


def _logsoftmax_terms(sp):
    mx = jnp.max(sp, axis=0)
    lz = jnp.log(jnp.sum(jnp.exp(sp - mx), axis=0)) + mx
    ls = sp - lz
    p = jnp.exp(ls)
    pls = jnp.sum(p * ls, axis=0)
    return ls, p, pls


def _stage1_kernel(sp_ref, tp_ref, klc_ref, dist_ref, ce_ref, scr_g, scr_t):
    sp = sp_ref[0]
    ls, p, pls = _logsoftmax_terms(sp)

    ls_c = ls[:, 1:_H + 1, 1:_W + 1]
    p_c = p[:, 1:_H + 1, 1:_W + 1]
    pls_c = pls[1:_H + 1, 1:_W + 1]

    kl_ud = pls_c - jnp.sum(p_c * ls[:, 2:_H + 2, 1:_W + 1], axis=0)
    kl_lr = pls_c - jnp.sum(p_c * ls[:, 1:_H + 1, 2:_W + 2], axis=0)
    klc_ref[0] = kl_ud + kl_lr

    t = tp_ref[0]
    tc = t[1:_H + 1, 1:_W + 1]
    gtb = (t[2:_H + 2, 1:_W + 1] != tc) | (t[1:_H + 1, 2:_W + 2] != tc) | (tc == _IGN)

    scr_g[...] = jnp.where(gtb, 0.0, _INF).astype(jnp.float32)

    def _fwd(i, carry):
        new = jnp.minimum(scr_g[pl.ds(i, 1), :], carry + 1.0)
        scr_g[pl.ds(i, 1), :] = new
        return new

    jax.lax.fori_loop(0, _H, _fwd, jnp.full((1, _W), _INF, jnp.float32))

    def _bwd(tt, carry):
        i = _H - 1 - tt
        new = jnp.minimum(scr_g[pl.ds(i, 1), :], carry + 1.0)
        scr_g[pl.ds(i, 1), :] = new
        return new

    jax.lax.fori_loop(0, _H, _bwd, jnp.full((1, _W), _INF, jnp.float32))

    g = scr_g[...]
    scr_t[...] = jnp.transpose(g * g)

    ys = jax.lax.broadcasted_iota(jnp.int32, (_W, _H), 0).astype(jnp.float32)

    def _env(j, d2t):
        row = scr_t[pl.ds(j, 1), :]
        off = (ys - j.astype(jnp.float32)) ** 2
        return jnp.minimum(d2t, row + off)

    d2t = jax.lax.fori_loop(0, _W, _env,
                            jnp.full((_W, _H), jnp.inf, jnp.float32))
    dist_ref[0] = jnp.transpose(jnp.maximum(jnp.sqrt(d2t) - 1.0, 0.0))

    lab = jnp.where(tc == _IGN, 0, tc)
    acc = jnp.zeros((_H, _W), jnp.float32)
    for c in range(_C):
        acc += jnp.where(lab == c, ls_c[c], 0.0)
    ce_ref[...] = jnp.sum(jnp.where(tc == _IGN, 0.0, -acc)).reshape(1, 1, 1)


def _stage2_kernel(sp_ref, klp_ref, dp_ref, eps_ref, lsce_ref, w_ref, cnt_ref):
    eps = eps_ref[0, 0]
    klp = klp_ref[0]
    binp = (klp > eps).astype(jnp.float32)
    acc = jnp.zeros((_H, _W), jnp.float32)
    for dx in (-1, 0, 1):
        for dy in (-1, 0, 1):
            acc += binp[1 + dx:_H + 1 + dx, 1 + dy:_W + 1 + dy]
    pred_b = acc > 0.0

    dp = dp_ref[0]
    d_c = dp[1:_H + 1, 1:_W + 1]
    best = dp[1 + _XR[0]:_H + 1 + _XR[0], 1 + _YR[0]:_W + 1 + _YR[0]]
    idx = jnp.zeros((_H, _W), jnp.int32)
    for k in range(1, 9):
        dk = dp[1 + _XR[k]:_H + 1 + _XR[k], 1 + _YR[k]:_W + 1 + _YR[k]]
        upd = dk < best
        idx = jnp.where(upd, k, idx)
        best = jnp.where(upd, dk, best)
    msel = pred_b & (idx != 8)

    sp = sp_ref[0]
    ls, p, pls = _logsoftmax_terms(sp)
    ls_c = ls[:, 1:_H + 1, 1:_W + 1]
    kls = []
    for k in range(8):
        dx, dy = _XR[k], _YR[k]
        pn = p[:, 1 + dx:_H + 1 + dx, 1 + dy:_W + 1 + dy]
        kls.append(pls[1 + dx:_H + 1 + dx, 1 + dy:_W + 1 + dy]
                   - jnp.sum(pn * ls_c, axis=0))

    m8 = kls[0]
    for k in range(1, 8):
        m8 = jnp.maximum(m8, kls[k])
    se = jnp.zeros_like(m8)
    ssum = jnp.zeros_like(m8)
    sat = jnp.zeros_like(m8)
    for k in range(8):
        se += jnp.exp(kls[k] - m8)
        ssum += kls[k]
        sat += jnp.where(idx == k, kls[k], 0.0)
    lz8 = jnp.log(se) + m8
    loss_px = -((_LS / 8.0) * (ssum - 8.0 * lz8) + (1.0 - _LS) * (sat - lz8))

    lsce_ref[...] = jnp.sum(jnp.where(msel, loss_px, 0.0)).reshape(1, 1, 1)
    w_ref[...] = jnp.sum(jnp.where(msel, jnp.minimum(d_c, _CLIP) / _CLIP, 0.0)).reshape(1, 1, 1)
    cnt_ref[...] = jnp.sum(pred_b.astype(jnp.float32)).reshape(1, 1, 1)


@jax.jit
def kernel(slices, targets):
    sp = jnp.pad(slices, ((0, 0), (0, 0), (1, 1), (1, 1)), mode='edge')
    tp = jnp.pad(targets[:, 0].astype(jnp.int32), ((0, 0), (1, 1), (1, 1)),
                 mode='edge')

    klc, dist, ce = pl.pallas_call(
        _stage1_kernel,
        grid=(_B,),
        in_specs=[
            pl.BlockSpec((1, _C, _H + 2, _W + 2), lambda b: (b, 0, 0, 0)),
            pl.BlockSpec((1, _H + 2, _W + 2), lambda b: (b, 0, 0)),
        ],
        out_specs=[
            pl.BlockSpec((1, _H, _W), lambda b: (b, 0, 0)),
            pl.BlockSpec((1, _H, _W), lambda b: (b, 0, 0)),
            pl.BlockSpec((1, 1, 1), lambda b: (b, 0, 0)),
        ],
        out_shape=[
            jax.ShapeDtypeStruct((_B, _H, _W), jnp.float32),
            jax.ShapeDtypeStruct((_B, _H, _W), jnp.float32),
            jax.ShapeDtypeStruct((_B, 1, 1), jnp.float32),
        ],
        scratch_shapes=[
            pltpu.VMEM((_H, _W), jnp.float32),
            pltpu.VMEM((_W, _H), jnp.float32),
        ],
    )(sp, tp)

    max_N = _H * _W * _MAXN
    eps = jax.lax.while_loop(
        lambda e: (klc > e).astype(jnp.float32).sum() > max_N,
        lambda e: e * 1.2,
        jnp.float32(1e-5),
    )

    klp = jnp.pad(klc, ((0, 0), (1, 1), (1, 1)), constant_values=-1.0)
    dpp = jnp.pad(dist, ((0, 0), (1, 1), (1, 1)), constant_values=1e5)
    lsce_s, w_s, cnt = pl.pallas_call(
        _stage2_kernel,
        grid=(_B,),
        in_specs=[
            pl.BlockSpec((1, _C, _H + 2, _W + 2), lambda b: (b, 0, 0, 0)),
            pl.BlockSpec((1, _H + 2, _W + 2), lambda b: (b, 0, 0)),
            pl.BlockSpec((1, _H + 2, _W + 2), lambda b: (b, 0, 0)),
            pl.BlockSpec((1, 1), lambda b: (0, 0)),
        ],
        out_specs=[pl.BlockSpec((1, 1, 1), lambda b: (b, 0, 0))] * 3,
        out_shape=[jax.ShapeDtypeStruct((_B, 1, 1), jnp.float32)] * 3,
    )(sp, klp, dpp, eps.reshape(1, 1))

    border = lsce_s.sum() * w_s.sum()
    border = jnp.where(cnt.sum() > 1.0, border, 0.0)
    return _BF * border + ce.sum()
